# baseline (speedup 1.0000x reference)
import functools

import jax
import jax.numpy as jnp
from jax.experimental import pallas as pl
from jax.experimental.pallas import tpu as pltpu

_HID = 768
_NE = 64
_K = 8
_NG = 8
_GK = 4
_GS = _NE // _NG
_NCAND = _NG * _GK
_AUX_ALPHA = 0.001
_Z_ALPHA = 0.0001

_TILE = 1024
_CHUNK = 128


def _gate_body(hs_ref, w_ref, idx_ref, wgt_ref, loss_ref, acc_probs, acc_counts,
               *, tokens, n_tiles):
    t = pl.program_id(0)

    @pl.when(t == 0)
    def _init():
        acc_probs[...] = jnp.zeros_like(acc_probs)
        acc_counts[...] = jnp.zeros_like(acc_counts)

    hs = hs_ref[...]
    w = w_ref[...]
    logits_full = jax.lax.dot_general(
        w, hs, (((1,), (1,)), ((), ())),
        preferred_element_type=jnp.float32)

    neg = jnp.float32(-jnp.inf)
    nc = _TILE // _CHUNK
    probs_acc = jnp.zeros((_NE, 1), jnp.float32)
    counts_acc = jnp.zeros((_NE, 1), jnp.float32)

    row_iota = jax.lax.broadcasted_iota(jnp.int32, (_NE, _CHUNK), 0)
    r8 = row_iota % _GS

    for c in range(nc):
        logits = logits_full[:, c * _CHUNK:(c + 1) * _CHUNK]

        rank = jnp.zeros((_NE, _CHUNK), jnp.int32)
        for s in range(1, _GS):
            left = jnp.concatenate([logits[s:, :], logits[:s, :]], axis=0)
            rs = _NE - (_GS - s)
            right = jnp.concatenate([logits[rs:, :], logits[:rs, :]], axis=0)
            within = r8 < (_GS - s)
            nb = jnp.where(within, left, right)
            beats = (nb > logits) | ((nb == logits) & ~within)
            rank = rank + beats.astype(jnp.int32)
        cand = rank < _GK

        work2 = jnp.where(cand, logits, neg)
        top_vals = []
        top_idx = []
        for _ in range(_K):
            m = jnp.max(work2, axis=0, keepdims=True)
            am = jnp.min(jnp.where(work2 == m, row_iota, _NE),
                         axis=0, keepdims=True)
            top_vals.append(m)
            top_idx.append(am)
            work2 = jnp.where(row_iota == am, neg, work2)
        tvals = jnp.concatenate(top_vals, axis=0)
        tidx = jnp.concatenate(top_idx, axis=0)

        ew = jnp.exp(tvals - tvals[0:1, :])
        wgt = ew / jnp.sum(ew, axis=0, keepdims=True)

        idx_ref[c * _CHUNK:(c + 1) * _CHUNK, :] = jnp.transpose(tidx)
        wgt_ref[c * _CHUNK:(c + 1) * _CHUNK, :] = jnp.transpose(wgt)

        fe = jnp.exp(logits - tvals[0:1, :])
        fs = jnp.sum(fe, axis=0, keepdims=True)
        probs_acc = probs_acc + jnp.sum(fe * (1.0 / fs), axis=1, keepdims=True)

        selmask = (cand & (work2 == neg)).astype(jnp.float32)
        counts_acc = counts_acc + jnp.sum(selmask, axis=1, keepdims=True)

    acc_probs[...] += probs_acc
    acc_counts[...] += counts_acc

    @pl.when(t == n_tiles - 1)
    def _finish():
        cnt = acc_counts[...]
        ps = acc_probs[...]
        tf = jnp.float32(tokens)
        aux = jnp.sum((cnt / tf) * (ps / tf)) * _AUX_ALPHA
        z = jnp.mean(jnp.log(ps) ** 2) * _Z_ALPHA
        loss_ref[...] = jnp.reshape(aux + z, (1, 1))


def kernel(hidden_states, gate_weight):
    tokens, hid = hidden_states.shape
    assert hid == _HID and gate_weight.shape == (_NE, _HID)
    assert tokens % _TILE == 0
    n_tiles = tokens // _TILE

    body = functools.partial(_gate_body, tokens=tokens, n_tiles=n_tiles)
    idx_t, wgt_t, loss = pl.pallas_call(
        body,
        grid=(n_tiles,),
        in_specs=[
            pl.BlockSpec((_TILE, _HID), lambda i: (i, 0)),
            pl.BlockSpec((_NE, _HID), lambda i: (0, 0)),
        ],
        out_specs=[
            pl.BlockSpec((_TILE, _K), lambda i: (i, 0)),
            pl.BlockSpec((_TILE, _K), lambda i: (i, 0)),
            pl.BlockSpec((1, 1), lambda i: (0, 0)),
        ],
        out_shape=[
            jax.ShapeDtypeStruct((tokens, _K), jnp.int32),
            jax.ShapeDtypeStruct((tokens, _K), jnp.float32),
            jax.ShapeDtypeStruct((1, 1), jnp.float32),
        ],
        scratch_shapes=[
            pltpu.VMEM((_NE, 1), jnp.float32),
            pltpu.VMEM((_NE, 1), jnp.float32),
        ],
    )(hidden_states, gate_weight)

    return idx_t, wgt_t, loss[0, 0]


# speedup vs baseline: 1.5207x; 1.5207x over previous
import functools

import jax
import jax.numpy as jnp
from jax.experimental import pallas as pl
from jax.experimental.pallas import tpu as pltpu

_HID = 768
_NE = 64
_K = 8
_NG = 8
_GK = 4
_GS = _NE // _NG
_NCAND = _NG * _GK
_AUX_ALPHA = 0.001
_Z_ALPHA = 0.0001

_TILE = 2048
_CHUNK = 128


def _gate_body(hs_ref, w_ref, idx_ref, wgt_ref, loss_ref, acc_probs, acc_counts,
               *, tokens, n_tiles):
    t = pl.program_id(0)

    @pl.when(t == 0)
    def _init():
        acc_probs[...] = jnp.zeros_like(acc_probs)
        acc_counts[...] = jnp.zeros_like(acc_counts)

    hs = hs_ref[...]
    w = w_ref[...]
    logits_full = jax.lax.dot_general(
        w, hs, (((1,), (1,)), ((), ())),
        preferred_element_type=jnp.float32)

    neg = jnp.float32(-jnp.inf)
    nc = _TILE // _CHUNK
    probs_acc = jnp.zeros((_NE, 1), jnp.float32)
    counts_acc = jnp.zeros((_NE, 1), jnp.float32)

    row_iota = jax.lax.broadcasted_iota(jnp.int32, (_NE, _CHUNK), 0)
    r8 = row_iota % _GS

    for c in range(nc):
        logits = logits_full[:, c * _CHUNK:(c + 1) * _CHUNK]

        rank = jnp.zeros((_NE, _CHUNK), jnp.int32)
        for s in range(1, _GS):
            left = jnp.concatenate([logits[s:, :], logits[:s, :]], axis=0)
            rs = _NE - (_GS - s)
            right = jnp.concatenate([logits[rs:, :], logits[:rs, :]], axis=0)
            within = r8 < (_GS - s)
            nb = jnp.where(within, left, right)
            beats = (nb > logits) | ((nb == logits) & ~within)
            rank = rank + beats.astype(jnp.int32)
        cand = rank < _GK

        work2 = jnp.where(cand, logits, neg)
        top_vals = []
        top_idx = []
        for _ in range(_K):
            m = jnp.max(work2, axis=0, keepdims=True)
            am = jnp.min(jnp.where(work2 == m, row_iota, _NE),
                         axis=0, keepdims=True)
            top_vals.append(m)
            top_idx.append(am)
            work2 = jnp.where(row_iota == am, neg, work2)
        tvals = jnp.concatenate(top_vals, axis=0)
        tidx = jnp.concatenate(top_idx, axis=0)

        ew = jnp.exp(tvals - tvals[0:1, :])
        wgt = ew / jnp.sum(ew, axis=0, keepdims=True)

        idx_ref[:, c * _CHUNK:(c + 1) * _CHUNK] = tidx
        wgt_ref[:, c * _CHUNK:(c + 1) * _CHUNK] = wgt

        fe = jnp.exp(logits - tvals[0:1, :])
        fs = jnp.sum(fe, axis=0, keepdims=True)
        probs_acc = probs_acc + jnp.sum(fe * (1.0 / fs), axis=1, keepdims=True)

        selmask = (cand & (work2 == neg)).astype(jnp.float32)
        counts_acc = counts_acc + jnp.sum(selmask, axis=1, keepdims=True)

    acc_probs[...] += probs_acc
    acc_counts[...] += counts_acc

    @pl.when(t == n_tiles - 1)
    def _finish():
        cnt = acc_counts[...]
        ps = acc_probs[...]
        tf = jnp.float32(tokens)
        aux = jnp.sum((cnt / tf) * (ps / tf)) * _AUX_ALPHA
        z = jnp.mean(jnp.log(ps) ** 2) * _Z_ALPHA
        loss_ref[...] = jnp.reshape(aux + z, (1, 1))


def kernel(hidden_states, gate_weight):
    tokens, hid = hidden_states.shape
    assert hid == _HID and gate_weight.shape == (_NE, _HID)
    assert tokens % _TILE == 0
    n_tiles = tokens // _TILE

    body = functools.partial(_gate_body, tokens=tokens, n_tiles=n_tiles)
    idx_t, wgt_t, loss = pl.pallas_call(
        body,
        grid=(n_tiles,),
        in_specs=[
            pl.BlockSpec((_TILE, _HID), lambda i: (i, 0)),
            pl.BlockSpec((_NE, _HID), lambda i: (0, 0)),
        ],
        out_specs=[
            pl.BlockSpec((_K, _TILE), lambda i: (0, i)),
            pl.BlockSpec((_K, _TILE), lambda i: (0, i)),
            pl.BlockSpec((1, 1), lambda i: (0, 0)),
        ],
        out_shape=[
            jax.ShapeDtypeStruct((_K, tokens), jnp.int32),
            jax.ShapeDtypeStruct((_K, tokens), jnp.float32),
            jax.ShapeDtypeStruct((1, 1), jnp.float32),
        ],
        scratch_shapes=[
            pltpu.VMEM((_NE, 1), jnp.float32),
            pltpu.VMEM((_NE, 1), jnp.float32),
        ],
    )(hidden_states, gate_weight)

    return idx_t.T, wgt_t.T, loss[0, 0]


# speedup vs baseline: 1.5672x; 1.0306x over previous
import functools

import jax
import jax.numpy as jnp
from jax.experimental import pallas as pl
from jax.experimental.pallas import tpu as pltpu

_HID = 768
_NE = 64
_K = 8
_NG = 8
_GK = 4
_GS = _NE // _NG
_NCAND = _NG * _GK
_AUX_ALPHA = 0.001
_Z_ALPHA = 0.0001

_TILE = 4096
_CHUNK = 128


def _gate_body(hs_ref, w_ref, idx_ref, wgt_ref, loss_ref, acc_probs, acc_counts,
               *, tokens, n_tiles):
    t = pl.program_id(0)

    @pl.when(t == 0)
    def _init():
        acc_probs[...] = jnp.zeros_like(acc_probs)
        acc_counts[...] = jnp.zeros_like(acc_counts)

    hs = hs_ref[...]
    w = w_ref[...]
    logits_full = jax.lax.dot_general(
        w, hs, (((1,), (1,)), ((), ())),
        preferred_element_type=jnp.float32)

    neg = jnp.float32(-jnp.inf)
    nc = _TILE // _CHUNK
    probs_acc = jnp.zeros((_NE, 1), jnp.float32)
    counts_acc = jnp.zeros((_NE, 1), jnp.float32)

    row_iota = jax.lax.broadcasted_iota(jnp.int32, (_NE, _CHUNK), 0)
    r8 = row_iota % _GS

    for c in range(nc):
        logits = logits_full[:, c * _CHUNK:(c + 1) * _CHUNK]

        rank = jnp.zeros((_NE, _CHUNK), jnp.int32)
        for s in range(1, _GS):
            left = jnp.concatenate([logits[s:, :], logits[:s, :]], axis=0)
            rs = _NE - (_GS - s)
            right = jnp.concatenate([logits[rs:, :], logits[:rs, :]], axis=0)
            within = r8 < (_GS - s)
            nb = jnp.where(within, left, right)
            beats = (nb > logits) | ((nb == logits) & ~within)
            rank = rank + beats.astype(jnp.int32)
        cand = rank < _GK

        work2 = jnp.where(cand, logits, neg)
        top_vals = []
        top_idx = []
        for _ in range(_K):
            m = jnp.max(work2, axis=0, keepdims=True)
            am = jnp.min(jnp.where(work2 == m, row_iota, _NE),
                         axis=0, keepdims=True)
            top_vals.append(m)
            top_idx.append(am)
            work2 = jnp.where(row_iota == am, neg, work2)
        tvals = jnp.concatenate(top_vals, axis=0)
        tidx = jnp.concatenate(top_idx, axis=0)

        ew = jnp.exp(tvals - tvals[0:1, :])
        wgt = ew / jnp.sum(ew, axis=0, keepdims=True)

        idx_ref[:, c * _CHUNK:(c + 1) * _CHUNK] = tidx
        wgt_ref[:, c * _CHUNK:(c + 1) * _CHUNK] = wgt

        fe = jnp.exp(logits - tvals[0:1, :])
        fs = jnp.sum(fe, axis=0, keepdims=True)
        probs_acc = probs_acc + jnp.sum(fe * (1.0 / fs), axis=1, keepdims=True)

        selmask = (cand & (work2 == neg)).astype(jnp.float32)
        counts_acc = counts_acc + jnp.sum(selmask, axis=1, keepdims=True)

    acc_probs[...] += probs_acc
    acc_counts[...] += counts_acc

    @pl.when(t == n_tiles - 1)
    def _finish():
        cnt = acc_counts[...]
        ps = acc_probs[...]
        tf = jnp.float32(tokens)
        aux = jnp.sum((cnt / tf) * (ps / tf)) * _AUX_ALPHA
        z = jnp.mean(jnp.log(ps) ** 2) * _Z_ALPHA
        loss_ref[...] = jnp.reshape(aux + z, (1, 1))


def kernel(hidden_states, gate_weight):
    tokens, hid = hidden_states.shape
    assert hid == _HID and gate_weight.shape == (_NE, _HID)
    assert tokens % _TILE == 0
    n_tiles = tokens // _TILE

    body = functools.partial(_gate_body, tokens=tokens, n_tiles=n_tiles)
    idx_t, wgt_t, loss = pl.pallas_call(
        body,
        grid=(n_tiles,),
        in_specs=[
            pl.BlockSpec((_TILE, _HID), lambda i: (i, 0)),
            pl.BlockSpec((_NE, _HID), lambda i: (0, 0)),
        ],
        out_specs=[
            pl.BlockSpec((_K, _TILE), lambda i: (0, i)),
            pl.BlockSpec((_K, _TILE), lambda i: (0, i)),
            pl.BlockSpec((1, 1), lambda i: (0, 0)),
        ],
        out_shape=[
            jax.ShapeDtypeStruct((_K, tokens), jnp.int32),
            jax.ShapeDtypeStruct((_K, tokens), jnp.float32),
            jax.ShapeDtypeStruct((1, 1), jnp.float32),
        ],
        scratch_shapes=[
            pltpu.VMEM((_NE, 1), jnp.float32),
            pltpu.VMEM((_NE, 1), jnp.float32),
        ],
    )(hidden_states, gate_weight)

    return idx_t.T, wgt_t.T, loss[0, 0]
